# baseline (speedup 1.0000x reference)
import jax
import jax.numpy as jnp
from jax import lax
from jax.experimental import pallas as pl
from jax.experimental.pallas import tpu as pltpu
from jax.experimental.pallas import tpu_sc as plsc

N = 5000
N_PAD = 5120
NUM_TECS = 16
NB = N_PAD // 16
ROWS_PER_TEC = N_PAD // NUM_TECS
SLOTS = 328
SCORE_THRESHOLD = 0.05
NMS_THRESHOLD = 0.5

_f32 = jnp.float32
_i32 = jnp.int32


def _nms_body(x1h, y1h, x2h, y2h, ph,
              o_x1, o_y1, o_x2, o_y2, o_p,
              vx1, vy1, vx2, vy2, vp,
              sx1, sy1, sx2, sy2, sa,
              kept_all, tmp96, pall, tmp_ax,
              vo0, vo1, vo2, vo3, vo4,
              sh_part):
    c = lax.axis_index("c")
    w = lax.axis_index("s")

    @pl.when(c == 0)
    def _():
        pltpu.sync_copy(x1h, vx1.at[pl.ds(0, N_PAD)])
        pltpu.sync_copy(y1h, vy1.at[pl.ds(0, N_PAD)])
        pltpu.sync_copy(x2h, vx2.at[pl.ds(0, N_PAD)])
        pltpu.sync_copy(y2h, vy2.at[pl.ds(0, N_PAD)])
        pltpu.sync_copy(ph, vp)

        zf = jnp.zeros((16,), _f32)

        @pl.loop(0, SLOTS)
        def _zero(k):
            sx1[pl.ds(16 * k, 16)] = zf
            sy1[pl.ds(16 * k, 16)] = zf
            sx2[pl.ds(16 * k, 16)] = zf
            sy2[pl.ds(16 * k, 16)] = zf
            sa[pl.ds(16 * k, 16)] = zf

        lanes = lax.iota(_i32, 16)

        def block_body(b, l_len):
            base = 16 * b
            cx1 = vx1[pl.ds(base, 16)]
            cy1 = vy1[pl.ds(base, 16)]
            cx2 = vx2[pl.ds(base, 16)]
            cy2 = vy2[pl.ds(base, 16)]
            cp = vp[pl.ds(base, 16)]
            carea = jnp.maximum(cx2 - cx1, 0.0) * jnp.maximum(cy2 - cy1, 0.0)

            xw1 = jnp.full((16,), vx1[pl.ds(base + w, 16)][0], _f32)
            yw1 = jnp.full((16,), vy1[pl.ds(base + w, 16)][0], _f32)
            xw2 = jnp.full((16,), vx2[pl.ds(base + w, 16)][0], _f32)
            yw2 = jnp.full((16,), vy2[pl.ds(base + w, 16)][0], _f32)
            aw = (jnp.maximum(xw2 - xw1, 0.0)
                  * jnp.maximum(yw2 - yw1, 0.0))

            nk = jnp.maximum(l_len - w + 15, 0) // 16
            nk4 = (nk + 3) // 4

            def chunk_body(j, sup):
                for u in range(4):
                    r0 = 16 * (4 * j + u)
                    rx1 = sx1[pl.ds(r0, 16)]
                    ry1 = sy1[pl.ds(r0, 16)]
                    rx2 = sx2[pl.ds(r0, 16)]
                    ry2 = sy2[pl.ds(r0, 16)]
                    ra = sa[pl.ds(r0, 16)]
                    xx1 = jnp.maximum(cx1, rx1)
                    yy1 = jnp.maximum(cy1, ry1)
                    xx2 = jnp.minimum(cx2, rx2)
                    yy2 = jnp.minimum(cy2, ry2)
                    inter = (jnp.maximum(xx2 - xx1, 0.0)
                             * jnp.maximum(yy2 - yy1, 0.0))
                    union = carea + ra - inter
                    sup = sup + jnp.where(inter > NMS_THRESHOLD * union,
                                          1.0, 0.0)
                return sup

            sup = lax.fori_loop(0, nk4, chunk_body,
                                jnp.zeros((16,), _f32))

            xx1 = jnp.maximum(cx1, xw1)
            yy1 = jnp.maximum(cy1, yw1)
            xx2 = jnp.minimum(cx2, xw2)
            yy2 = jnp.minimum(cy2, yw2)
            inter = (jnp.maximum(xx2 - xx1, 0.0)
                     * jnp.maximum(yy2 - yy1, 0.0))
            union = carea + aw - inter
            wv = jnp.full((16,), w, _i32)
            rowm = (jnp.where(inter > NMS_THRESHOLD * union, 1.0, 0.0)
                    * jnp.where(lanes > wv, 1.0, 0.0))

            tmp96[pl.ds(0, 16)] = sup
            tmp96[pl.ds(16, 16)] = rowm
            pltpu.sync_copy(tmp96.at[pl.ds(0, 32)],
                            sh_part.at[pl.ds(32 * w, 32)])
            plsc.subcore_barrier()

            pltpu.sync_copy(sh_part, pall)
            acc = pall[pl.ds(0, 16)]
            for i in range(1, NUM_TECS):
                acc = acc + pall[pl.ds(32 * i, 16)]
            keptf = jnp.where((cp >= SCORE_THRESHOLD) & (acc == 0.0),
                              1.0, 0.0)
            j_w = (w - (l_len & 15)) & 15
            pos = w * 0
            tw = w * 0
            for t in range(16):
                rowt = pall[pl.ds(32 * t + 16, 16)]
                ktf = keptf[t]
                keptf = keptf * (1.0 - rowt * jnp.full((16,), ktf, _f32))
                kti = ktf.astype(_i32)
                tw = jnp.where((pos == j_w) & (kti != 0), t, tw)
                pos = pos + kti
            kept_all[pl.ds(base, 16)] = jnp.where(keptf != 0.0, 1, 0)

            @pl.when(j_w < pos)
            def _():
                slot = (l_len + j_w - w) // 16
                s0 = 16 * slot
                nx1 = jnp.full((16,), vx1[pl.ds(base + tw, 16)][0], _f32)
                ny1 = jnp.full((16,), vy1[pl.ds(base + tw, 16)][0], _f32)
                nx2 = jnp.full((16,), vx2[pl.ds(base + tw, 16)][0], _f32)
                ny2 = jnp.full((16,), vy2[pl.ds(base + tw, 16)][0], _f32)
                sx1[pl.ds(s0, 16)] = nx1
                sy1[pl.ds(s0, 16)] = ny1
                sx2[pl.ds(s0, 16)] = nx2
                sy2[pl.ds(s0, 16)] = ny2
                sa[pl.ds(s0, 16)] = (jnp.maximum(nx2 - nx1, 0.0)
                                     * jnp.maximum(ny2 - ny1, 0.0))

            return l_len + pos

        lax.fori_loop(0, NB, block_body, jnp.int32(0))

        row0 = ROWS_PER_TEC * w
        for g in range(ROWS_PER_TEC // 16):
            idx = row0 + 16 * g
            keepf = jnp.where(kept_all[pl.ds(idx, 16)] != 0, 1.0, 0.0)
            vo0[pl.ds(16 * g, 16)] = vx1[pl.ds(idx, 16)] * keepf
            vo1[pl.ds(16 * g, 16)] = vy1[pl.ds(idx, 16)] * keepf
            vo2[pl.ds(16 * g, 16)] = vx2[pl.ds(idx, 16)] * keepf
            vo3[pl.ds(16 * g, 16)] = vy2[pl.ds(idx, 16)] * keepf
            vo4[pl.ds(16 * g, 16)] = vp[pl.ds(idx, 16)] * keepf
        pltpu.sync_copy(vo0, o_x1.at[pl.ds(row0, ROWS_PER_TEC)])
        pltpu.sync_copy(vo1, o_y1.at[pl.ds(row0, ROWS_PER_TEC)])
        pltpu.sync_copy(vo2, o_x2.at[pl.ds(row0, ROWS_PER_TEC)])
        pltpu.sync_copy(vo3, o_y2.at[pl.ds(row0, ROWS_PER_TEC)])
        pltpu.sync_copy(vo4, o_p.at[pl.ds(row0, ROWS_PER_TEC)])


@jax.jit
def kernel(boxes, scores):
    probs = jax.nn.sigmoid(scores)
    order = jnp.argsort(-probs)
    b = jnp.take(boxes, order, axis=0)
    p = jnp.take(probs, order, axis=0)

    pad = N_PAD - N
    x1 = jnp.pad(b[:, 0], (0, pad))
    y1 = jnp.pad(b[:, 1], (0, pad))
    x2 = jnp.pad(b[:, 2], (0, pad))
    y2 = jnp.pad(b[:, 3], (0, pad))
    pp = jnp.pad(p, (0, pad))

    mesh = plsc.VectorSubcoreMesh(core_axis_name="c", subcore_axis_name="s")
    f = pl.kernel(
        _nms_body,
        out_type=[jax.ShapeDtypeStruct((N_PAD,), _f32)] * 5,
        mesh=mesh,
        scratch_types=[
            pltpu.VMEM((N_PAD + 16,), _f32),
            pltpu.VMEM((N_PAD + 16,), _f32),
            pltpu.VMEM((N_PAD + 16,), _f32),
            pltpu.VMEM((N_PAD + 16,), _f32),
            pltpu.VMEM((N_PAD,), _f32),
            pltpu.VMEM((SLOTS * 16,), _f32),
            pltpu.VMEM((SLOTS * 16,), _f32),
            pltpu.VMEM((SLOTS * 16,), _f32),
            pltpu.VMEM((SLOTS * 16,), _f32),
            pltpu.VMEM((SLOTS * 16,), _f32),
            pltpu.VMEM((N_PAD,), _i32),
            pltpu.VMEM((96,), _f32),
            pltpu.VMEM((NUM_TECS * 32,), _f32),
            pltpu.VMEM((160,), _f32),
            pltpu.VMEM((ROWS_PER_TEC,), _f32),
            pltpu.VMEM((ROWS_PER_TEC,), _f32),
            pltpu.VMEM((ROWS_PER_TEC,), _f32),
            pltpu.VMEM((ROWS_PER_TEC,), _f32),
            pltpu.VMEM((ROWS_PER_TEC,), _f32),
            pltpu.VMEM_SHARED((NUM_TECS * 32,), _f32),
        ],
    )
    o_x1, o_y1, o_x2, o_y2, o_p = f(x1, y1, x2, y2, pp)
    out = jnp.stack([o_x1, o_y1, o_x2, o_y2, o_p], axis=1)
    return out[:N]
